# Initial kernel scaffold; baseline (speedup 1.0000x reference)
import jax
import jax.numpy as jnp
from jax.experimental import pallas as pl


def kernel(x, edge_index, W0, as0, ad0, b0, fcw0, fcb0, W1, as1, ad1, b1, fcw1, fcb1, fow, fob):
    raise NotImplementedError("write your pallas kernel here")


# speedup vs baseline: 21.5131x; 21.5131x over previous
import dataclasses
import functools

import jax
import jax.numpy as jnp
from jax import lax
from jax.experimental import pallas as pl
from jax.experimental.pallas import tpu as pltpu
from jax.experimental.pallas import tpu_sc as plsc

N = 10000
D = 128
E = 320000

NC = 2
NS = 16
L = 16
NW = NC * NS
EPT = E // NW
K = 80
NCHUNK = EPT // K
RPT = 624
TAIL = N - NS * RPT

R = 400


def _pre_body(x_ref, w_ref, avs_ref, avd_ref, h_ref, as_ref, ad_ref):
    h = jnp.dot(x_ref[...], w_ref[...], preferred_element_type=jnp.float32)
    h_ref[...] = h
    as_ref[...] = jnp.sum(h * avs_ref[...], axis=1, keepdims=True)
    ad_ref[...] = jnp.sum(h * avd_ref[...], axis=1, keepdims=True)


def _tc_pre(x, W, a_s, a_d):
    return pl.pallas_call(
        _pre_body,
        grid=(N // R,),
        in_specs=[
            pl.BlockSpec((R, D), lambda i: (i, 0)),
            pl.BlockSpec((D, D), lambda i: (0, 0)),
            pl.BlockSpec((1, D), lambda i: (0, 0)),
            pl.BlockSpec((1, D), lambda i: (0, 0)),
        ],
        out_specs=[
            pl.BlockSpec((R, D), lambda i: (i, 0)),
            pl.BlockSpec((R, 1), lambda i: (i, 0)),
            pl.BlockSpec((R, 1), lambda i: (i, 0)),
        ],
        out_shape=[
            jax.ShapeDtypeStruct((N, D), jnp.float32),
            jax.ShapeDtypeStruct((N, 1), jnp.float32),
            jax.ShapeDtypeStruct((N, 1), jnp.float32),
        ],
    )(x, W, a_s.reshape(1, D), a_d.reshape(1, D))


def _post_body(ah_ref, dpart_ref, h_ref, as_ref, advr_ref, c_ref, b_ref,
               fcw_ref, fcb_ref, o_ref):
    z = as_ref[...] + advr_ref[...]
    e = jnp.maximum(z, 0.2 * z)
    exs = jnp.exp(e - c_ref[0, 0])
    S = ah_ref[0] + ah_ref[1] + exs * h_ref[...]
    den = dpart_ref[...] + exs + 1e-16
    y = S / den + b_ref[...]
    y = jnp.where(y > 0, y, jnp.exp(jnp.minimum(y, 0.0)) - 1.0)
    o_ref[...] = (jnp.dot(y, fcw_ref[...], preferred_element_type=jnp.float32)
                  + fcb_ref[...])


def _tc_post(acch, den2d, h, as2d, ad2d, c2d, b, fcw, fcb):
    return pl.pallas_call(
        _post_body,
        grid=(N // R,),
        in_specs=[
            pl.BlockSpec((NC, R, D), lambda i: (0, i, 0)),
            pl.BlockSpec((R, 1), lambda i: (i, 0)),
            pl.BlockSpec((R, D), lambda i: (i, 0)),
            pl.BlockSpec((R, 1), lambda i: (i, 0)),
            pl.BlockSpec((R, 1), lambda i: (i, 0)),
            pl.BlockSpec((1, 1), lambda i: (0, 0)),
            pl.BlockSpec((1, D), lambda i: (0, 0)),
            pl.BlockSpec((D, D), lambda i: (0, 0)),
            pl.BlockSpec((1, D), lambda i: (0, 0)),
        ],
        out_specs=pl.BlockSpec((R, D), lambda i: (i, 0)),
        out_shape=jax.ShapeDtypeStruct((N, D), jnp.float32),
    )(acch, den2d, h, as2d, ad2d, c2d, b.reshape(1, D), fcw, fcb.reshape(1, D))


def _final_body(h_ref, w_ref, b_ref, g_ref, o_ref):
    z = (jnp.dot(h_ref[...], w_ref[...], preferred_element_type=jnp.float32)
         + b_ref[...] + g_ref[...])
    m = jnp.max(z, axis=1, keepdims=True)
    p = jnp.exp(z - m)
    o_ref[...] = p / jnp.sum(p, axis=1, keepdims=True)


def _tc_final(h, fow, fob, g):
    return pl.pallas_call(
        _final_body,
        grid=(N // R,),
        in_specs=[
            pl.BlockSpec((R, D), lambda i: (i, 0)),
            pl.BlockSpec((D, D), lambda i: (0, 0)),
            pl.BlockSpec((1, D), lambda i: (0, 0)),
            pl.BlockSpec((R, D), lambda i: (i, 0)),
        ],
        out_specs=pl.BlockSpec((R, D), lambda i: (i, 0)),
        out_shape=jax.ShapeDtypeStruct((N, D), jnp.float32),
    )(h, fow, fob.reshape(1, D), g)


def _sc_edge(h, asv, adv, cvec, src3, dst3, zh, zd):
    mesh = plsc.VectorSubcoreMesh(core_axis_name="c", subcore_axis_name="s")
    cp = pltpu.CompilerParams()
    if "needs_layout_passes" in pltpu.CompilerParams.__dataclass_fields__:
        cp = dataclasses.replace(cp, needs_layout_passes=False)

    @functools.partial(
        pl.kernel,
        mesh=mesh,
        compiler_params=cp,
        out_type=[
            jax.ShapeDtypeStruct((NC * N, D), jnp.float32),
            jax.ShapeDtypeStruct((NW, N), jnp.float32),
        ],
        scratch_types=[
            pltpu.VMEM((N,), jnp.float32),
            pltpu.VMEM((N,), jnp.float32),
            pltpu.VMEM((L,), jnp.float32),
            pltpu.VMEM((K,), jnp.int32),
            pltpu.VMEM((K,), jnp.int32),
            pltpu.VMEM((K,), jnp.float32),
            pltpu.VMEM((K, D), jnp.float32),
            pltpu.VMEM((N,), jnp.float32),
            pltpu.VMEM_SHARED((N, D), jnp.float32),
            pltpu.SemaphoreType.DMA,
        ],
    )
    def k(h_hbm, asv_hbm, adv_hbm, c_hbm, src_hbm, dst_hbm, zh_hbm, zd_hbm,
          outh_hbm, outd_hbm,
          asv_v, adv_v, c_v, src_v, dst_v, ex_v, rows_v, denom_v,
          acc_h, sem):
        cid = lax.axis_index("c")
        sid = lax.axis_index("s")
        wid = sid * NC + cid

        pltpu.sync_copy(asv_hbm, asv_v)
        pltpu.sync_copy(adv_hbm, adv_v)
        pltpu.sync_copy(c_hbm, c_v)
        pltpu.sync_copy(zd_hbm, denom_v)
        pltpu.sync_copy(zh_hbm.at[pl.ds(sid * RPT, RPT)],
                        acc_h.at[pl.ds(sid * RPT, RPT)])

        @pl.when(sid == NS - 1)
        def _():
            pltpu.sync_copy(zh_hbm.at[pl.ds(NS * RPT, TAIL)],
                            acc_h.at[pl.ds(NS * RPT, TAIL)])

        plsc.subcore_barrier()

        cvec_r = c_v[...]
        lane = lax.iota(jnp.int32, 16)

        @pl.loop(0, NCHUNK)
        def _(g):
            pltpu.sync_copy(src_hbm.at[wid, g], src_v)
            pltpu.sync_copy(dst_hbm.at[wid, g], dst_v)
            pltpu.async_copy(h_hbm.at[src_v], rows_v, sem).wait()
            for j in range(K // L):
                s16 = src_v[pl.ds(j * L, L)]
                d16 = dst_v[pl.ds(j * L, L)]
                z = (plsc.load_gather(asv_v, [s16])
                     + plsc.load_gather(adv_v, [d16]))
                e = jnp.maximum(z, 0.2 * z)
                ex = jnp.exp(e - cvec_r)
                ex_v[pl.ds(j * L, L)] = ex
                plsc.addupdate_scatter(denom_v, [d16], ex)

            @pl.loop(0, K)
            def _(i):
                exb = plsc.load_gather(ex_v, [lane * 0 + i])
                for kk in range(D // L):
                    rows_v[i, pl.ds(kk * L, L)] = (
                        rows_v[i, pl.ds(kk * L, L)] * exb)

            pltpu.sync_copy(rows_v, acc_h.at[dst_v], add=True)

        plsc.subcore_barrier()
        pltpu.sync_copy(acc_h.at[pl.ds(sid * RPT, RPT)],
                        outh_hbm.at[pl.ds(cid * N + sid * RPT, RPT)])
        pltpu.sync_copy(denom_v, outd_hbm.at[wid])

        @pl.when(sid == NS - 1)
        def _():
            pltpu.sync_copy(acc_h.at[pl.ds(NS * RPT, TAIL)],
                            outh_hbm.at[pl.ds(cid * N + NS * RPT, TAIL)])

    return k(h, asv, adv, cvec, src3, dst3, zh, zd)


def _layer(x, src3, dst3, zh, zd, W, a_s, a_d, b, fcw, fcb):
    h, as2d, ad2d = _tc_pre(x, W, a_s, a_d)
    asv = as2d.reshape(N)
    adv = ad2d.reshape(N)
    m = jnp.max(asv) + jnp.max(adv)
    c = jnp.maximum(m, 0.2 * m)
    cvec = jnp.full((L,), 0.0, jnp.float32) + c
    outh, outd = _sc_edge(h, asv, adv, cvec, src3, dst3, zh, zd)
    acch = outh.reshape(NC, N, D)
    den2d = jnp.sum(outd, axis=0).reshape(N, 1)
    return _tc_post(acch, den2d, h, as2d, ad2d, c.reshape(1, 1), b, fcw, fcb)


def kernel(x, edge_index, W0, as0, ad0, b0, fcw0, fcb0,
           W1, as1, ad1, b1, fcw1, fcb1, fow, fob):
    ei = edge_index.astype(jnp.int32)
    src3 = ei[0].reshape(NW, NCHUNK, K)
    dst3 = ei[1].reshape(NW, NCHUNK, K)
    zh = jnp.zeros((N, D), jnp.float32)
    zd = jnp.zeros((N,), jnp.float32)

    h1 = _layer(x, src3, dst3, zh, zd, W0, as0, ad0, b0, fcw0, fcb0)
    h2 = _layer(h1, src3, dst3, zh, zd, W1, as1, ad1, b1, fcw1, fcb1)

    u = jax.random.uniform(jax.random.key(42), (N, D), jnp.float32)
    g = -jnp.log(-jnp.log(u + 1e-20) + 1e-20)
    return _tc_final(h2, fow, fob, g)


# speedup vs baseline: 22.0505x; 1.0250x over previous
import dataclasses
import functools

import jax
import jax.numpy as jnp
from jax import lax
from jax.experimental import pallas as pl
from jax.experimental.pallas import tpu as pltpu
from jax.experimental.pallas import tpu_sc as plsc

N = 10000
D = 128
E = 320000

NC = 2
NS = 16
L = 16
NW = NC * NS
EPT = E // NW
K = 80
NCHUNK = EPT // K
RPT = 624
TAIL = N - NS * RPT

R = 400


def _pre_body(x_ref, w_ref, avs_ref, avd_ref, h_ref, as_ref, ad_ref):
    h = jnp.dot(x_ref[...], w_ref[...], preferred_element_type=jnp.float32)
    h_ref[...] = h
    as_ref[...] = jnp.sum(h * avs_ref[...], axis=1, keepdims=True)
    ad_ref[...] = jnp.sum(h * avd_ref[...], axis=1, keepdims=True)


def _tc_pre(x, W, a_s, a_d):
    return pl.pallas_call(
        _pre_body,
        grid=(N // R,),
        in_specs=[
            pl.BlockSpec((R, D), lambda i: (i, 0)),
            pl.BlockSpec((D, D), lambda i: (0, 0)),
            pl.BlockSpec((1, D), lambda i: (0, 0)),
            pl.BlockSpec((1, D), lambda i: (0, 0)),
        ],
        out_specs=[
            pl.BlockSpec((R, D), lambda i: (i, 0)),
            pl.BlockSpec((R, 1), lambda i: (i, 0)),
            pl.BlockSpec((R, 1), lambda i: (i, 0)),
        ],
        out_shape=[
            jax.ShapeDtypeStruct((N, D), jnp.float32),
            jax.ShapeDtypeStruct((N, 1), jnp.float32),
            jax.ShapeDtypeStruct((N, 1), jnp.float32),
        ],
    )(x, W, a_s.reshape(1, D), a_d.reshape(1, D))


def _post_body(ah_ref, dpart_ref, h_ref, as_ref, advr_ref, c_ref, b_ref,
               fcw_ref, fcb_ref, o_ref):
    z = as_ref[...] + advr_ref[...]
    e = jnp.maximum(z, 0.2 * z)
    exs = jnp.exp(e - c_ref[0, 0])
    S = ah_ref[0] + ah_ref[1] + exs * h_ref[...]
    den = dpart_ref[...] + exs + 1e-16
    y = S / den + b_ref[...]
    y = jnp.where(y > 0, y, jnp.exp(jnp.minimum(y, 0.0)) - 1.0)
    o_ref[...] = (jnp.dot(y, fcw_ref[...], preferred_element_type=jnp.float32)
                  + fcb_ref[...])


def _tc_post(acch, den2d, h, as2d, ad2d, c2d, b, fcw, fcb):
    return pl.pallas_call(
        _post_body,
        grid=(N // R,),
        in_specs=[
            pl.BlockSpec((NC, R, D), lambda i: (0, i, 0)),
            pl.BlockSpec((R, 1), lambda i: (i, 0)),
            pl.BlockSpec((R, D), lambda i: (i, 0)),
            pl.BlockSpec((R, 1), lambda i: (i, 0)),
            pl.BlockSpec((R, 1), lambda i: (i, 0)),
            pl.BlockSpec((1, 1), lambda i: (0, 0)),
            pl.BlockSpec((1, D), lambda i: (0, 0)),
            pl.BlockSpec((D, D), lambda i: (0, 0)),
            pl.BlockSpec((1, D), lambda i: (0, 0)),
        ],
        out_specs=pl.BlockSpec((R, D), lambda i: (i, 0)),
        out_shape=jax.ShapeDtypeStruct((N, D), jnp.float32),
    )(acch, den2d, h, as2d, ad2d, c2d, b.reshape(1, D), fcw, fcb.reshape(1, D))


def _final_body(h_ref, w_ref, b_ref, g_ref, o_ref):
    z = (jnp.dot(h_ref[...], w_ref[...], preferred_element_type=jnp.float32)
         + b_ref[...] + g_ref[...])
    m = jnp.max(z, axis=1, keepdims=True)
    p = jnp.exp(z - m)
    o_ref[...] = p / jnp.sum(p, axis=1, keepdims=True)


def _tc_final(h, fow, fob, g):
    return pl.pallas_call(
        _final_body,
        grid=(N // R,),
        in_specs=[
            pl.BlockSpec((R, D), lambda i: (i, 0)),
            pl.BlockSpec((D, D), lambda i: (0, 0)),
            pl.BlockSpec((1, D), lambda i: (0, 0)),
            pl.BlockSpec((R, D), lambda i: (i, 0)),
        ],
        out_specs=pl.BlockSpec((R, D), lambda i: (i, 0)),
        out_shape=jax.ShapeDtypeStruct((N, D), jnp.float32),
    )(h, fow, fob.reshape(1, D), g)


def _sc_edge(h, asv, adv, cvec, src3, dst3, zh, zd):
    mesh = plsc.VectorSubcoreMesh(core_axis_name="c", subcore_axis_name="s")
    cp = pltpu.CompilerParams()
    if "needs_layout_passes" in pltpu.CompilerParams.__dataclass_fields__:
        cp = dataclasses.replace(cp, needs_layout_passes=False)

    @functools.partial(
        pl.kernel,
        mesh=mesh,
        compiler_params=cp,
        out_type=[
            jax.ShapeDtypeStruct((NC * N, D), jnp.float32),
            jax.ShapeDtypeStruct((NW, N), jnp.float32),
        ],
        scratch_types=[
            pltpu.VMEM((N,), jnp.float32),
            pltpu.VMEM((N,), jnp.float32),
            pltpu.VMEM((L,), jnp.float32),
            pltpu.VMEM((K,), jnp.int32),
            pltpu.VMEM((K,), jnp.int32),
            pltpu.VMEM((K,), jnp.float32),
            pltpu.VMEM((K, D), jnp.float32),
            pltpu.VMEM((N,), jnp.float32),
            pltpu.VMEM_SHARED((N, D), jnp.float32),
            pltpu.SemaphoreType.DMA,
        ],
    )
    def k(h_hbm, asv_hbm, adv_hbm, c_hbm, src_hbm, dst_hbm, zh_hbm, zd_hbm,
          outh_hbm, outd_hbm,
          asv_v, adv_v, c_v, src_v, dst_v, ex_v, rows_v,
          denom_v, acc_h, sem):
        cid = lax.axis_index("c")
        sid = lax.axis_index("s")
        wid = sid * NC + cid

        pltpu.sync_copy(asv_hbm, asv_v)
        pltpu.sync_copy(adv_hbm, adv_v)
        pltpu.sync_copy(c_hbm, c_v)
        pltpu.sync_copy(zd_hbm, denom_v)
        pltpu.sync_copy(zh_hbm.at[pl.ds(sid * RPT, RPT)],
                        acc_h.at[pl.ds(sid * RPT, RPT)])

        @pl.when(sid == NS - 1)
        def _():
            pltpu.sync_copy(zh_hbm.at[pl.ds(NS * RPT, TAIL)],
                            acc_h.at[pl.ds(NS * RPT, TAIL)])

        plsc.subcore_barrier()

        cvec_r = c_v[...]
        lane = lax.iota(jnp.int32, 16)

        @pl.loop(0, NCHUNK)
        def _(g):
            pltpu.sync_copy(src_hbm.at[wid, g], src_v)
            pltpu.sync_copy(dst_hbm.at[wid, g], dst_v)
            pltpu.make_async_copy(h_hbm.at[src_v], rows_v, sem).start()
            for j in range(K // L):
                s16 = src_v[pl.ds(j * L, L)]
                d16 = dst_v[pl.ds(j * L, L)]
                z = (plsc.load_gather(asv_v, [s16])
                     + plsc.load_gather(adv_v, [d16]))
                e = jnp.maximum(z, 0.2 * z)
                ex = jnp.exp(e - cvec_r)
                ex_v[pl.ds(j * L, L)] = ex
                plsc.addupdate_scatter(denom_v, [d16], ex)

            pltpu.make_async_copy(h_hbm.at[src_v], rows_v, sem).wait()

            @pl.loop(0, K)
            def _(i):
                exb = plsc.load_gather(ex_v, [lane * 0 + i])
                for kk in range(D // L):
                    rows_v[i, pl.ds(kk * L, L)] = (
                        rows_v[i, pl.ds(kk * L, L)] * exb)

            pltpu.sync_copy(rows_v, acc_h.at[dst_v], add=True)

        plsc.subcore_barrier()
        pltpu.sync_copy(acc_h.at[pl.ds(sid * RPT, RPT)],
                        outh_hbm.at[pl.ds(cid * N + sid * RPT, RPT)])
        pltpu.sync_copy(denom_v, outd_hbm.at[wid])

        @pl.when(sid == NS - 1)
        def _():
            pltpu.sync_copy(acc_h.at[pl.ds(NS * RPT, TAIL)],
                            outh_hbm.at[pl.ds(cid * N + NS * RPT, TAIL)])

    return k(h, asv, adv, cvec, src3, dst3, zh, zd)


def _layer(x, src3, dst3, zh, zd, W, a_s, a_d, b, fcw, fcb):
    h, as2d, ad2d = _tc_pre(x, W, a_s, a_d)
    asv = as2d.reshape(N)
    adv = ad2d.reshape(N)
    m = jnp.max(asv) + jnp.max(adv)
    c = jnp.maximum(m, 0.2 * m)
    cvec = jnp.full((L,), 0.0, jnp.float32) + c
    outh, outd = _sc_edge(h, asv, adv, cvec, src3, dst3, zh, zd)
    acch = outh.reshape(NC, N, D)
    den2d = jnp.sum(outd, axis=0).reshape(N, 1)
    return _tc_post(acch, den2d, h, as2d, ad2d, c.reshape(1, 1), b, fcw, fcb)


def kernel(x, edge_index, W0, as0, ad0, b0, fcw0, fcb0,
           W1, as1, ad1, b1, fcw1, fcb1, fow, fob):
    ei = edge_index.astype(jnp.int32)
    src3 = ei[0].reshape(NW, NCHUNK, K)
    dst3 = ei[1].reshape(NW, NCHUNK, K)
    zh = jnp.zeros((N, D), jnp.float32)
    zd = jnp.zeros((N,), jnp.float32)

    h1 = _layer(x, src3, dst3, zh, zd, W0, as0, ad0, b0, fcw0, fcb0)
    h2 = _layer(h1, src3, dst3, zh, zd, W1, as1, ad1, b1, fcw1, fcb1)

    u = jax.random.uniform(jax.random.key(42), (N, D), jnp.float32)
    g = -jnp.log(-jnp.log(u + 1e-20) + 1e-20)
    return _tc_final(h2, fow, fob, g)


# speedup vs baseline: 30.9276x; 1.4026x over previous
import dataclasses
import functools

import jax
import jax.numpy as jnp
from jax import lax
from jax.experimental import pallas as pl
from jax.experimental.pallas import tpu as pltpu
from jax.experimental.pallas import tpu_sc as plsc

N = 10000
D = 128
E = 320000

NC = 2
NS = 16
L = 16
NW = NC * NS
EPT = E // NW
K = 64
NCHUNK = (EPT + K - 1) // K
PADE = NCHUNK * K - EPT
NP = N + 16
RPT = 624
TAIL = N - NS * RPT

R = 400


def _pre_body(x_ref, w_ref, avs_ref, avd_ref, h_ref, as_ref, ad_ref):
    h = jnp.dot(x_ref[...], w_ref[...], preferred_element_type=jnp.float32)
    h_ref[...] = h
    as_ref[...] = jnp.sum(h * avs_ref[...], axis=1, keepdims=True)
    ad_ref[...] = jnp.sum(h * avd_ref[...], axis=1, keepdims=True)


def _tc_pre(x, W, a_s, a_d):
    return pl.pallas_call(
        _pre_body,
        grid=(N // R,),
        in_specs=[
            pl.BlockSpec((R, D), lambda i: (i, 0)),
            pl.BlockSpec((D, D), lambda i: (0, 0)),
            pl.BlockSpec((1, D), lambda i: (0, 0)),
            pl.BlockSpec((1, D), lambda i: (0, 0)),
        ],
        out_specs=[
            pl.BlockSpec((R, D), lambda i: (i, 0)),
            pl.BlockSpec((R, 1), lambda i: (i, 0)),
            pl.BlockSpec((R, 1), lambda i: (i, 0)),
        ],
        out_shape=[
            jax.ShapeDtypeStruct((N, D), jnp.float32),
            jax.ShapeDtypeStruct((N, 1), jnp.float32),
            jax.ShapeDtypeStruct((N, 1), jnp.float32),
        ],
    )(x, W, a_s.reshape(1, D), a_d.reshape(1, D))


def _post_body(ah_ref, dpart_ref, h_ref, as_ref, advr_ref, c_ref, b_ref,
               fcw_ref, fcb_ref, o_ref):
    z = as_ref[...] + advr_ref[...]
    e = jnp.maximum(z, 0.2 * z)
    exs = jnp.exp(e - c_ref[0, 0])
    S = ah_ref[0] + ah_ref[1] + exs * h_ref[...]
    den = dpart_ref[...] + exs + 1e-16
    y = S / den + b_ref[...]
    y = jnp.where(y > 0, y, jnp.exp(jnp.minimum(y, 0.0)) - 1.0)
    o_ref[...] = (jnp.dot(y, fcw_ref[...], preferred_element_type=jnp.float32)
                  + fcb_ref[...])


def _tc_post(acch, den2d, h, as2d, ad2d, c2d, b, fcw, fcb):
    return pl.pallas_call(
        _post_body,
        grid=(N // R,),
        in_specs=[
            pl.BlockSpec((NC, R, D), lambda i: (0, i, 0)),
            pl.BlockSpec((R, 1), lambda i: (i, 0)),
            pl.BlockSpec((R, D), lambda i: (i, 0)),
            pl.BlockSpec((R, 1), lambda i: (i, 0)),
            pl.BlockSpec((R, 1), lambda i: (i, 0)),
            pl.BlockSpec((1, 1), lambda i: (0, 0)),
            pl.BlockSpec((1, D), lambda i: (0, 0)),
            pl.BlockSpec((D, D), lambda i: (0, 0)),
            pl.BlockSpec((1, D), lambda i: (0, 0)),
        ],
        out_specs=pl.BlockSpec((R, D), lambda i: (i, 0)),
        out_shape=jax.ShapeDtypeStruct((N, D), jnp.float32),
    )(acch, den2d, h, as2d, ad2d, c2d, b.reshape(1, D), fcw, fcb.reshape(1, D))


def _final_body(h_ref, w_ref, b_ref, g_ref, o_ref):
    z = (jnp.dot(h_ref[...], w_ref[...], preferred_element_type=jnp.float32)
         + b_ref[...] + g_ref[...])
    m = jnp.max(z, axis=1, keepdims=True)
    p = jnp.exp(z - m)
    o_ref[...] = p / jnp.sum(p, axis=1, keepdims=True)


def _tc_final(h, fow, fob, g):
    return pl.pallas_call(
        _final_body,
        grid=(N // R,),
        in_specs=[
            pl.BlockSpec((R, D), lambda i: (i, 0)),
            pl.BlockSpec((D, D), lambda i: (0, 0)),
            pl.BlockSpec((1, D), lambda i: (0, 0)),
            pl.BlockSpec((R, D), lambda i: (i, 0)),
        ],
        out_specs=pl.BlockSpec((R, D), lambda i: (i, 0)),
        out_shape=jax.ShapeDtypeStruct((N, D), jnp.float32),
    )(h, fow, fob.reshape(1, D), g)


def _sc_edge(h, asv, adv, cvec, sd3, zh, zd):
    mesh = plsc.VectorSubcoreMesh(core_axis_name="c", subcore_axis_name="s")
    cp = pltpu.CompilerParams()
    if "needs_layout_passes" in pltpu.CompilerParams.__dataclass_fields__:
        cp = dataclasses.replace(cp, needs_layout_passes=False)

    @functools.partial(
        pl.kernel,
        mesh=mesh,
        compiler_params=cp,
        out_type=[
            jax.ShapeDtypeStruct((NC * N, D), jnp.float32),
            jax.ShapeDtypeStruct((NW, NP), jnp.float32),
        ],
        scratch_types=[
            pltpu.VMEM((NP,), jnp.float32),
            pltpu.VMEM((NP,), jnp.float32),
            pltpu.VMEM((L,), jnp.float32),
            pltpu.VMEM((2, K), jnp.int32),
            pltpu.VMEM((2, K), jnp.int32),
            pltpu.VMEM((K,), jnp.int32),
            pltpu.VMEM((K,), jnp.int32),
            pltpu.VMEM((K,), jnp.float32),
            pltpu.VMEM((K, D), jnp.float32),
            pltpu.VMEM((K, D), jnp.float32),
            pltpu.VMEM((NP,), jnp.float32),
            pltpu.VMEM_SHARED((NP, D), jnp.float32),
            pltpu.SemaphoreType.DMA,
            pltpu.SemaphoreType.DMA,
            pltpu.SemaphoreType.DMA,
            pltpu.SemaphoreType.DMA,
            pltpu.SemaphoreType.DMA,
            pltpu.SemaphoreType.DMA,
        ],
    )
    def k(h_hbm, asv_hbm, adv_hbm, c_hbm, sd_hbm, zh_hbm, zd_hbm,
          outh_hbm, outd_hbm,
          asv_v, adv_v, c_v, sd_a, sd_b, dsc_a, dsc_b, ex_v, rows_a, rows_b,
          denom_v, acc_h, isem_a, isem_b, gsem_a, gsem_b, ssem_a, ssem_b):
        cid = lax.axis_index("c")
        sid = lax.axis_index("s")
        wid = sid * NC + cid

        pltpu.sync_copy(asv_hbm, asv_v.at[pl.ds(0, N)])
        pltpu.sync_copy(adv_hbm, adv_v.at[pl.ds(0, N)])
        pltpu.sync_copy(c_hbm, c_v)
        zero16 = jnp.zeros((L,), jnp.float32)
        asv_v[pl.ds(N, L)] = zero16
        adv_v[pl.ds(N, L)] = zero16

        @pl.loop(0, NP, step=L)
        def _(i):
            denom_v[pl.ds(i, L)] = zero16

        @pl.loop(0, RPT, step=104)
        def _(rr):
            pltpu.sync_copy(zh_hbm.at[pl.ds(sid * RPT + rr, 104)],
                            acc_h.at[pl.ds(sid * RPT + rr, 104)])

        @pl.when(sid == NS - 1)
        def _():
            pltpu.sync_copy(zh_hbm.at[pl.ds(NS * RPT, TAIL)],
                            acc_h.at[pl.ds(NS * RPT, TAIL)])

        plsc.subcore_barrier()

        cvec_r = c_v[...]

        def body(g, sd_m, dsc_m, rows_m, isem_m, gsem_m, ssem_m,
                 sd_o, dsc_o, rows_o, isem_o, gsem_o, ssem_o):
            @pl.when(g >= 1)
            def _():
                pltpu.make_async_copy(rows_o, acc_h.at[dsc_o], ssem_o).wait()

            @pl.when(jnp.logical_and(g + 1 >= 0, g + 1 <= NCHUNK - 1))
            def _():
                pltpu.make_async_copy(sd_hbm.at[wid, g + 1], sd_o,
                                      isem_o).wait()
                pltpu.make_async_copy(h_hbm.at[sd_o.at[0]], rows_o,
                                      gsem_o).start()

            @pl.when(jnp.logical_and(g >= 0, g <= NCHUNK - 1))
            def _():
                for j in range(K // L):
                    s16 = sd_m[0, pl.ds(j * L, L)]
                    d16 = sd_m[1, pl.ds(j * L, L)]
                    dsc_m[pl.ds(j * L, L)] = d16
                    z = (plsc.load_gather(asv_v, [s16])
                         + plsc.load_gather(adv_v, [d16]))
                    e = jnp.maximum(z, 0.2 * z)
                    ex = jnp.exp(e - cvec_r)
                    ex_v[pl.ds(j * L, L)] = ex
                    plsc.addupdate_scatter(denom_v, [d16], ex)

                pltpu.make_async_copy(h_hbm.at[sd_m.at[0]], rows_m,
                                      gsem_m).wait()
                lane = lax.iota(jnp.int32, 16)

                @pl.loop(0, K)
                def _(i):
                    exb = plsc.load_gather(ex_v, [lane * 0 + i])
                    for kk in range(D // L):
                        rows_m[i, pl.ds(kk * L, L)] = (
                            rows_m[i, pl.ds(kk * L, L)] * exb)

                pltpu.make_async_copy(rows_m, acc_h.at[dsc_m],
                                      ssem_m).start()

            @pl.when(jnp.logical_and(g + 2 >= 0, g + 2 <= NCHUNK - 1))
            def _():
                pltpu.make_async_copy(sd_hbm.at[wid, g + 2], sd_m,
                                      isem_m).start()

        aset = (sd_a, dsc_a, rows_a, isem_a, gsem_a, ssem_a)
        bset = (sd_b, dsc_b, rows_b, isem_b, gsem_b, ssem_b)

        @pl.loop(0, (NCHUNK + 3) // 2)
        def _(r):
            g = 2 * r - 2
            body(g, *aset, *bset)
            body(g + 1, *bset, *aset)

        plsc.subcore_barrier()

        @pl.loop(0, RPT, step=104)
        def _(rr):
            pltpu.sync_copy(acc_h.at[pl.ds(sid * RPT + rr, 104)],
                            outh_hbm.at[pl.ds(cid * N + sid * RPT + rr, 104)])

        pltpu.sync_copy(denom_v, outd_hbm.at[wid])

        @pl.when(sid == NS - 1)
        def _():
            pltpu.sync_copy(acc_h.at[pl.ds(NS * RPT, TAIL)],
                            outh_hbm.at[pl.ds(cid * N + NS * RPT, TAIL)])

    return k(h, asv, adv, cvec, sd3, zh, zd)


def _layer(x, sd3, zh, zd, W, a_s, a_d, b, fcw, fcb):
    h, as2d, ad2d = _tc_pre(x, W, a_s, a_d)
    asv = as2d.reshape(N)
    adv = ad2d.reshape(N)
    m = jnp.max(asv) + jnp.max(adv)
    c = jnp.maximum(m, 0.2 * m)
    cvec = jnp.full((L,), 0.0, jnp.float32) + c
    outh, outd = _sc_edge(h, asv, adv, cvec, sd3, zh, zd)
    acch = outh.reshape(NC, N, D)
    den2d = jnp.sum(outd[:, :N], axis=0).reshape(N, 1)
    return _tc_post(acch, den2d, h, as2d, ad2d, c.reshape(1, 1), b, fcw, fcb)


def kernel(x, edge_index, W0, as0, ad0, b0, fcw0, fcb0,
           W1, as1, ad1, b1, fcw1, fcb1, fow, fob):
    ei = edge_index.astype(jnp.int32)
    srcp = jnp.concatenate(
        [ei[0].reshape(NW, EPT), jnp.zeros((NW, PADE), jnp.int32)], axis=1)
    dstp = jnp.concatenate(
        [ei[1].reshape(NW, EPT), jnp.full((NW, PADE), N, jnp.int32)], axis=1)
    sd3 = jnp.stack([srcp.reshape(NW, NCHUNK, K),
                     dstp.reshape(NW, NCHUNK, K)], axis=2)
    zh = jnp.zeros((N, D), jnp.float32)
    zd = jnp.zeros((N,), jnp.float32)

    h1 = _layer(x, sd3, zh, zd, W0, as0, ad0, b0, fcw0, fcb0)
    h2 = _layer(h1, sd3, zh, zd, W1, as1, ad1, b1, fcw1, fcb1)

    u = jax.random.uniform(jax.random.key(42), (N, D), jnp.float32)
    g = -jnp.log(-jnp.log(u + 1e-20) + 1e-20)
    return _tc_final(h2, fow, fob, g)
